# Initial kernel scaffold; baseline (speedup 1.0000x reference)
import jax
import jax.numpy as jnp
from jax.experimental import pallas as pl


def kernel(input_ids, embed_table, pos_table):
    raise NotImplementedError("write your pallas kernel here")


# speedup vs baseline: 2.6589x; 2.6589x over previous
import functools
import math

import jax
import jax.numpy as jnp
from jax import lax
from jax.experimental import pallas as pl
from jax.experimental.pallas import tpu as pltpu
from jax.experimental.pallas import tpu_sc as plsc

D = 128
S = 512
N = 1024 * S
NC = 2
NS = 16
NW = NC * NS
ROWS_W = N // NW
CH = 64
NCH = ROWS_W // CH
LANES = 16
SCALE = math.sqrt(float(D))

_mesh = plsc.VectorSubcoreMesh(core_axis_name="c", subcore_axis_name="s")


@functools.partial(
    pl.kernel,
    mesh=_mesh,
    out_type=jax.ShapeDtypeStruct((N, D), jnp.float32),
    scratch_types=[
        pltpu.VMEM((S, D), jnp.float32),
        pltpu.VMEM((ROWS_W,), jnp.int32),
        pltpu.VMEM((2, CH, D), jnp.float32),
        pltpu.VMEM((2, CH, D), jnp.float32),
        pltpu.SemaphoreType.DMA,
        pltpu.SemaphoreType.DMA,
        pltpu.SemaphoreType.DMA,
        pltpu.SemaphoreType.DMA,
    ],
)
def _emb(ids_hbm, table_hbm, pos_hbm, out_hbm,
         pos_v, idx_v, rows_v, outs_v, gs0, gs1, ss0, ss1):
    wid = lax.axis_index("s") * NC + lax.axis_index("c")
    base = wid * ROWS_W
    pltpu.sync_copy(pos_hbm, pos_v)
    pltpu.sync_copy(ids_hbm.at[pl.ds(base, ROWS_W)], idx_v)

    gsems = (gs0, gs1)
    ssems = (ss0, ss1)

    def start_gather(c, b):
        pltpu.async_copy(
            table_hbm.at[idx_v.at[pl.ds(c * CH, CH)]],
            rows_v.at[b], gsems[b])

    def wait_gather(b):
        pltpu.make_async_copy(
            table_hbm.at[idx_v.at[pl.ds(0, CH)]],
            rows_v.at[b], gsems[b]).wait()

    def start_scatter(c, b):
        pltpu.async_copy(
            outs_v.at[b],
            out_hbm.at[pl.ds(base + c * CH, CH)], ssems[b])

    def wait_scatter(b):
        pltpu.make_async_copy(
            outs_v.at[b],
            out_hbm.at[pl.ds(0, CH)], ssems[b]).wait()

    def compute(c, b):
        pbase = lax.rem(c, S // CH) * CH

        def row_body(r, carry):
            pr = pbase + r
            for j in range(D // LANES):
                sl = pl.ds(j * LANES, LANES)
                outs_v[b, r, sl] = rows_v[b, r, sl] * SCALE + pos_v[pr, sl]
            return carry

        lax.fori_loop(0, CH, row_body, 0)

    start_gather(0, 0)
    start_gather(1, 1)

    def pair_body(i, carry):
        c0 = i * 2
        for b in range(2):
            cc = c0 + b
            wait_gather(b)

            @pl.when(i > 0)
            def _():
                wait_scatter(b)

            compute(cc, b)
            start_scatter(cc, b)

            @pl.when(cc + 2 < NCH)
            def _():
                start_gather(cc + 2, b)
        return carry

    lax.fori_loop(0, NCH // 2, pair_body, 0)
    wait_scatter(0)
    wait_scatter(1)


def kernel(input_ids, embed_table, pos_table):
    bsz, seq = input_ids.shape
    ids = input_ids.reshape(-1).astype(jnp.int32)
    out = _emb(ids, embed_table, pos_table)
    return out.reshape(bsz, seq, D)
